# baseline (speedup 1.0000x reference)
import functools

import jax
import jax.numpy as jnp
from jax import lax
from jax.experimental import pallas as pl
from jax.experimental.pallas import tpu as pltpu
from jax.experimental.pallas import tpu_sc as plsc

N = 10000
D = 128
ED = 4
L = 3
NC = 2
NS = 16
NW = NC * NS
K = 128
NPAD = 10112
RPT = NPAD // NS

_HI = jax.lax.Precision.HIGHEST


def _mm(a, b):
    return jax.lax.dot_general(a, b, (((1,), (0,)), ((), ())),
                               preferred_element_type=jnp.float32,
                               precision=_HI)


def _sc_segsum_gather(table, idx2, nb, z128):
    mesh = plsc.VectorSubcoreMesh(core_axis_name="c", subcore_axis_name="s")

    @functools.partial(
        pl.kernel,
        out_type=jax.ShapeDtypeStruct((NC, NPAD, D), jnp.float32),
        mesh=mesh,
        scratch_types=[
            pltpu.VMEM((2, K), jnp.int32),
            pltpu.VMEM((2, K), jnp.int32),
            pltpu.VMEM((K, D), jnp.float32),
            pltpu.VMEM((K, D), jnp.float32),
            pltpu.VMEM_SHARED((NPAD, D), jnp.float32),
            pltpu.SemaphoreType.DMA,
            pltpu.SemaphoreType.DMA,
        ],
    )
    def k(table_h, idx_h, z_h, out_h, ixa, ixb, rowsa, rowsb, acc, sema,
          semb):
        c = lax.axis_index("c")
        s = lax.axis_index("s")
        pltpu.sync_copy(z_h, acc.at[pl.ds(s * RPT, RPT)])
        w = c * NS + s
        b0 = w * nb

        ga = lambda: pltpu.make_async_copy(table_h.at[ixa.at[0]], rowsa, sema)
        gb = lambda: pltpu.make_async_copy(table_h.at[ixb.at[0]], rowsb, semb)

        pltpu.sync_copy(idx_h.at[b0], ixa)
        ga().start()
        plsc.subcore_barrier()

        @pl.loop(0, nb, step=2)
        def _(j):
            pltpu.sync_copy(idx_h.at[b0 + j + 1], ixb)
            ga().wait()
            gb().start()
            pltpu.sync_copy(rowsa, acc.at[ixa.at[1]], add=True)
            pltpu.sync_copy(idx_h.at[b0 + j + 2], ixa)
            gb().wait()
            ga().start()
            pltpu.sync_copy(rowsb, acc.at[ixb.at[1]], add=True)

        ga().wait()
        plsc.subcore_barrier()
        pltpu.sync_copy(acc.at[pl.ds(s * RPT, RPT)],
                        out_h.at[c, pl.ds(s * RPT, RPT)])

    return k(table, idx2, z128)


def _sc_segsum_rows(rows_tab, idx2, nb, z128):
    mesh = plsc.VectorSubcoreMesh(core_axis_name="c", subcore_axis_name="s")

    @functools.partial(
        pl.kernel,
        out_type=jax.ShapeDtypeStruct((NC, NPAD, D), jnp.float32),
        mesh=mesh,
        scratch_types=[
            pltpu.VMEM((2, K), jnp.int32),
            pltpu.VMEM((2, K), jnp.int32),
            pltpu.VMEM((K, D), jnp.float32),
            pltpu.VMEM((K, D), jnp.float32),
            pltpu.VMEM_SHARED((NPAD, D), jnp.float32),
            pltpu.SemaphoreType.DMA,
            pltpu.SemaphoreType.DMA,
        ],
    )
    def k(rows_h, idx_h, z_h, out_h, ixa, ixb, rowsa, rowsb, acc, sema,
          semb):
        c = lax.axis_index("c")
        s = lax.axis_index("s")
        pltpu.sync_copy(z_h, acc.at[pl.ds(s * RPT, RPT)])
        w = c * NS + s
        b0 = w * nb

        def la(j):
            return pltpu.make_async_copy(
                rows_h.at[pl.ds((b0 + j) * K, K)], rowsa, sema)

        def lb(j):
            return pltpu.make_async_copy(
                rows_h.at[pl.ds((b0 + j) * K, K)], rowsb, semb)

        pltpu.sync_copy(idx_h.at[b0], ixa)
        la(0).start()
        plsc.subcore_barrier()

        @pl.loop(0, nb, step=2)
        def _(j):
            pltpu.sync_copy(idx_h.at[b0 + j + 1], ixb)
            la(j).wait()
            lb(j + 1).start()
            pltpu.sync_copy(rowsa, acc.at[ixa.at[1]], add=True)
            pltpu.sync_copy(idx_h.at[b0 + j + 2], ixa)
            lb(j + 1).wait()
            la(j + 2).start()
            pltpu.sync_copy(rowsb, acc.at[ixb.at[1]], add=True)

        la(nb).wait()
        plsc.subcore_barrier()
        pltpu.sync_copy(acc.at[pl.ds(s * RPT, RPT)],
                        out_h.at[c, pl.ds(s * RPT, RPT)])

    return k(rows_tab, idx2, z128)


_BR = 1000
_GRID = N // _BR


def _tc_wprep(Wn, bn, We, be, Wm, bm):
    def body(wn, bn_, we, be_, wm, bm_, wa, ba_o, ce):
        for l in range(L):
            wm1 = wm[l, :D, :]
            wm2 = wm[l, D:, :]
            wa[l] = _mm(wn[l], wm1)
            ba_o[l] = _mm(bn_[l][None, :], wm1)
            row0 = _mm(be_[l][None, :], wm2) + bm_[l][None, :]
            wep = _mm(we[l], wm2)
            ce[l] = jnp.concatenate(
                [row0, wep, jnp.zeros((16 - 1 - ED, D), jnp.float32)], axis=0)

    return pl.pallas_call(
        body,
        out_shape=[
            jax.ShapeDtypeStruct((L, D, D), jnp.float32),
            jax.ShapeDtypeStruct((L, 1, D), jnp.float32),
            jax.ShapeDtypeStruct((L, 16, D), jnp.float32),
        ],
    )(Wn, bn, We, be, Wm, bm)


def _tc_init(x, Wp, bp, Wa0, ba0, st0, st1):
    def body(x_b, wp, bp_, wa, ba_, s0_b, s1_b, h_o, a_o, s_o):
        h = _mm(x_b[...], wp[...]) + bp_[...]
        h_o[...] = h
        a_o[...] = _mm(h, wa[...]) + ba_[...]
        s_o[...] = s0_b[...] + s1_b[...]

    full = lambda s: pl.BlockSpec(s, lambda i: (0,) * len(s))
    row = lambda c: pl.BlockSpec((_BR, c), lambda i: (i, 0))
    return pl.pallas_call(
        body,
        grid=(_GRID,),
        in_specs=[row(D), full((D, D)), full((1, D)), full((D, D)),
                  full((1, D)), row(16), row(16)],
        out_specs=[row(D), row(D), row(16)],
        out_shape=[
            jax.ShapeDtypeStruct((N, D), jnp.float32),
            jax.ShapeDtypeStruct((N, D), jnp.float32),
            jax.ShapeDtypeStruct((N, 16), jnp.float32),
        ],
    )(x, Wp, bp, Wa0, ba0, st0, st1)


def _tc_layer(h, a0, a1, s16, ce, g, b, wa_n, ba_n, last):
    def body(h_b, a0_b, a1_b, s_b, ce_, g_, b_, *rest):
        if last:
            (h_o,) = rest
        else:
            wa, ba_, h_o, a_o = rest
        agg = a0_b[...] + a1_b[...] + _mm(s_b[...], ce_[...])
        mu = jnp.mean(agg, axis=1, keepdims=True)
        xc = agg - mu
        var = jnp.mean(xc * xc, axis=1, keepdims=True)
        hln = xc * jax.lax.rsqrt(var + 1e-5) * g_[...] + b_[...]
        hn = h_b[...] + jnp.maximum(hln, 0.0)
        h_o[...] = hn
        if not last:
            a_o[...] = _mm(hn, wa[...]) + ba_[...]

    full = lambda s: pl.BlockSpec(s, lambda i: (0,) * len(s))
    row = lambda c: pl.BlockSpec((_BR, c), lambda i: (i, 0))
    in_specs = [row(D), row(D), row(D), row(16), full((16, D)),
                full((1, D)), full((1, D))]
    out_specs = [row(D)]
    out_shape = [jax.ShapeDtypeStruct((N, D), jnp.float32)]
    args = [h, a0, a1, s16, ce, g, b]
    if not last:
        in_specs += [full((D, D)), full((1, D))]
        out_specs += [row(D)]
        out_shape += [jax.ShapeDtypeStruct((N, D), jnp.float32)]
        args += [wa_n, ba_n]
    out = pl.pallas_call(
        body, grid=(_GRID,), in_specs=in_specs,
        out_specs=out_specs, out_shape=out_shape,
    )(*args)
    return out if not last else (out[0], None)


def kernel(x, edge_index, edge_attr, Wp, bp, Wn, bn, We, be, Wm, bm, lng, lnb):
    src = edge_index[0]
    dst = edge_index[1]
    E = src.shape[0]
    per_w = -(-E // NW)
    nb = -(-per_w // K)
    nb = nb + (nb % 2)
    P = nb * K
    EP = P * NW
    nbt = EP // K
    pad = EP - E
    srcp = jnp.concatenate([src, jnp.zeros((pad,), jnp.int32)])
    dstp = jnp.concatenate([dst, jnp.full((pad,), N, jnp.int32)])
    idx2 = jnp.stack([srcp.reshape(nbt, K), dstp.reshape(nbt, K)], axis=1)
    idx2 = jnp.concatenate([idx2, jnp.zeros((1, 2, K), jnp.int32)], axis=0)
    ea128 = jnp.concatenate(
        [jnp.ones((E, 1), jnp.float32), edge_attr,
         jnp.zeros((E, D - 1 - ED), jnp.float32)], axis=1)
    ea128 = jnp.concatenate(
        [ea128, jnp.zeros((pad + K, D), jnp.float32)], axis=0)
    z128 = jnp.zeros((RPT, D), jnp.float32)

    Wa, ba, Ce = _tc_wprep(Wn, bn, We, be, Wm, bm)
    stats = _sc_segsum_rows(ea128, idx2, nb, z128)
    h, A, S16 = _tc_init(x, Wp, bp.reshape(1, D), Wa[0], ba[0],
                         stats[0, :N, :16], stats[1, :N, :16])
    for l in range(L):
        acc = _sc_segsum_gather(A, idx2, nb, z128)
        last = l == L - 1
        h, A = _tc_layer(
            h, acc[0, :N], acc[1, :N], S16, Ce[l],
            lng[l].reshape(1, D), lnb[l].reshape(1, D),
            None if last else Wa[l + 1], None if last else ba[l + 1], last)
    return h


# speedup vs baseline: 1.2689x; 1.2689x over previous
import functools

import jax
import jax.numpy as jnp
from jax import lax
from jax.experimental import pallas as pl
from jax.experimental.pallas import tpu as pltpu
from jax.experimental.pallas import tpu_sc as plsc

N = 10000
D = 128
ED = 4
L = 3
NC = 2
NS = 16
NW = NC * NS
K = 64
NPAD = 10112
RPT = NPAD // NS
H = NPAD // 2
NACC = 5120
APT = NPAD // NS
ZPT = NACC // NS

_HI = jax.lax.Precision.HIGHEST


def _mm(a, b):
    return jax.lax.dot_general(a, b, (((1,), (0,)), ((), ())),
                               preferred_element_type=jnp.float32,
                               precision=_HI)


def _sc_segsum_gather(table, idx2, z128):
    nbt = idx2.shape[0] - 1
    nbc = nbt // NS
    mesh = plsc.VectorSubcoreMesh(core_axis_name="c", subcore_axis_name="s")

    @functools.partial(
        pl.kernel,
        out_type=jax.ShapeDtypeStruct((NC, NACC, D), jnp.float32),
        mesh=mesh,
        scratch_types=[
            pltpu.VMEM((2, K), jnp.int32),
            pltpu.VMEM((2, K), jnp.int32),
            pltpu.VMEM((K,), jnp.int32),
            pltpu.VMEM((K, D), jnp.float32),
            pltpu.VMEM_SHARED((NPAD, D), jnp.float32),
            pltpu.VMEM_SHARED((NACC, D), jnp.float32),
            pltpu.SemaphoreType.DMA,
            pltpu.SemaphoreType.DMA,
            pltpu.SemaphoreType.DMA,
        ],
    )
    def k(table_h, idx_h, z_h, out_h, ixa, ixb, ixd, rows, tbl, acc,
          sema, semb, semg):
        c = lax.axis_index("c")
        s = lax.axis_index("s")
        b0 = s * nbc
        la = lambda j: pltpu.make_async_copy(idx_h.at[b0 + j], ixa, sema)
        lb = lambda j: pltpu.make_async_copy(idx_h.at[b0 + j], ixb, semb)
        la(0).start()
        pltpu.sync_copy(table_h.at[pl.ds(s * APT, APT)],
                        tbl.at[pl.ds(s * APT, APT)])
        pltpu.sync_copy(z_h.at[pl.ds(0, ZPT)], acc.at[pl.ds(s * ZPT, ZPT)])
        la(0).wait()
        plsc.subcore_barrier()
        cH = c * H

        def process(ix):
            for g in range(K // 16):
                lv = ix[1, pl.ds(g * 16, 16)] - cH
                m = (lv >= 0) & (lv < H)
                ixd[pl.ds(g * 16, 16)] = jnp.where(m, lv, H)
            pltpu.async_copy(tbl.at[ix.at[0]], rows, semg).wait()
            pltpu.sync_copy(rows, acc.at[ixd], add=True)

        @pl.loop(0, nbc, step=2)
        def _(j):
            lb(j + 1).start()
            process(ixa)
            lb(j + 1).wait()
            la(j + 2).start()
            process(ixb)
            la(j + 2).wait()

        plsc.subcore_barrier()
        pltpu.sync_copy(acc.at[pl.ds(s * ZPT, ZPT)],
                        out_h.at[c, pl.ds(s * ZPT, ZPT)])

    return k(table, idx2, z128)


def _sc_segsum_rows(rows_tab, idx2, nb, z128):
    mesh = plsc.VectorSubcoreMesh(core_axis_name="c", subcore_axis_name="s")

    @functools.partial(
        pl.kernel,
        out_type=jax.ShapeDtypeStruct((NC, NPAD, D), jnp.float32),
        mesh=mesh,
        scratch_types=[
            pltpu.VMEM((2, K), jnp.int32),
            pltpu.VMEM((2, K), jnp.int32),
            pltpu.VMEM((K, D), jnp.float32),
            pltpu.VMEM((K, D), jnp.float32),
            pltpu.VMEM_SHARED((NPAD, D), jnp.float32),
            pltpu.SemaphoreType.DMA,
            pltpu.SemaphoreType.DMA,
        ],
    )
    def k(rows_h, idx_h, z_h, out_h, ixa, ixb, rowsa, rowsb, acc, sema,
          semb):
        c = lax.axis_index("c")
        s = lax.axis_index("s")
        pltpu.sync_copy(z_h, acc.at[pl.ds(s * RPT, RPT)])
        w = c * NS + s
        b0 = w * nb

        def la(j):
            return pltpu.make_async_copy(
                rows_h.at[pl.ds((b0 + j) * K, K)], rowsa, sema)

        def lb(j):
            return pltpu.make_async_copy(
                rows_h.at[pl.ds((b0 + j) * K, K)], rowsb, semb)

        pltpu.sync_copy(idx_h.at[b0], ixa)
        la(0).start()
        plsc.subcore_barrier()

        @pl.loop(0, nb, step=2)
        def _(j):
            pltpu.sync_copy(idx_h.at[b0 + j + 1], ixb)
            la(j).wait()
            lb(j + 1).start()
            pltpu.sync_copy(rowsa, acc.at[ixa.at[1]], add=True)
            pltpu.sync_copy(idx_h.at[b0 + j + 2], ixa)
            lb(j + 1).wait()
            la(j + 2).start()
            pltpu.sync_copy(rowsb, acc.at[ixb.at[1]], add=True)

        la(nb).wait()
        plsc.subcore_barrier()
        pltpu.sync_copy(acc.at[pl.ds(s * RPT, RPT)],
                        out_h.at[c, pl.ds(s * RPT, RPT)])

    return k(rows_tab, idx2, z128)


_BR = 1000
_GRID = N // _BR


def _tc_wprep(Wn, bn, We, be, Wm, bm):
    def body(wn, bn_, we, be_, wm, bm_, wa, ba_o, ce):
        for l in range(L):
            wm1 = wm[l, :D, :]
            wm2 = wm[l, D:, :]
            wa[l] = _mm(wn[l], wm1)
            ba_o[l] = _mm(bn_[l][None, :], wm1)
            row0 = _mm(be_[l][None, :], wm2) + bm_[l][None, :]
            wep = _mm(we[l], wm2)
            ce[l] = jnp.concatenate(
                [row0, wep, jnp.zeros((16 - 1 - ED, D), jnp.float32)], axis=0)

    return pl.pallas_call(
        body,
        out_shape=[
            jax.ShapeDtypeStruct((L, D, D), jnp.float32),
            jax.ShapeDtypeStruct((L, 1, D), jnp.float32),
            jax.ShapeDtypeStruct((L, 16, D), jnp.float32),
        ],
    )(Wn, bn, We, be, Wm, bm)


def _tc_init(x, Wp, bp, Wa0, ba0):
    def body(x_b, wp, bp_, wa, ba_, h_o, a_o):
        h = _mm(x_b[...], wp[...]) + bp_[...]
        h_o[...] = h
        a_o[...] = _mm(h, wa[...]) + ba_[...]

    full = lambda s: pl.BlockSpec(s, lambda i: (0,) * len(s))
    row = lambda c: pl.BlockSpec((_BR, c), lambda i: (i, 0))
    return pl.pallas_call(
        body,
        grid=(_GRID,),
        in_specs=[row(D), full((D, D)), full((1, D)), full((D, D)),
                  full((1, D))],
        out_specs=[row(D), row(D)],
        out_shape=[
            jax.ShapeDtypeStruct((N, D), jnp.float32),
            jax.ShapeDtypeStruct((NPAD, D), jnp.float32),
        ],
    )(x, Wp, bp, Wa0, ba0)


def _tc_layer(h, agg_in, st0, st1, ce, g, b, wa_n, ba_n, last):
    def body(h_b, a_b, s0_b, s1_b, ce_, g_, b_, *rest):
        if last:
            (h_o,) = rest
        else:
            wa, ba_, h_o, a_o = rest
        s = s0_b[...] + s1_b[...]
        agg = a_b[...] + _mm(s, ce_[...])
        mu = jnp.mean(agg, axis=1, keepdims=True)
        xc = agg - mu
        var = jnp.mean(xc * xc, axis=1, keepdims=True)
        hln = xc * jax.lax.rsqrt(var + 1e-5) * g_[...] + b_[...]
        hn = h_b[...] + jnp.maximum(hln, 0.0)
        h_o[...] = hn
        if not last:
            a_o[...] = _mm(hn, wa[...]) + ba_[...]

    full = lambda s: pl.BlockSpec(s, lambda i: (0,) * len(s))
    row = lambda c: pl.BlockSpec((_BR, c), lambda i: (i, 0))
    in_specs = [row(D), row(D), row(16), row(16), full((16, D)),
                full((1, D)), full((1, D))]
    out_specs = [row(D)]
    out_shape = [jax.ShapeDtypeStruct((N, D), jnp.float32)]
    args = [h, agg_in, st0, st1, ce, g, b]
    if not last:
        in_specs += [full((D, D)), full((1, D))]
        out_specs += [row(D)]
        out_shape += [jax.ShapeDtypeStruct((NPAD, D), jnp.float32)]
        args += [wa_n, ba_n]
    out = pl.pallas_call(
        body, grid=(_GRID,), in_specs=in_specs,
        out_specs=out_specs, out_shape=out_shape,
    )(*args)
    return out if not last else (out[0], None)


def kernel(x, edge_index, edge_attr, Wp, bp, Wn, bn, We, be, Wm, bm, lng, lnb):
    src = edge_index[0]
    dst = edge_index[1]
    E = src.shape[0]
    per_w = -(-E // NW)
    nb = -(-per_w // K)
    nb = nb + (nb % 2)
    P = nb * K
    EP = P * NW
    nbt = EP // K
    pad = EP - E
    srcp = jnp.concatenate([src, jnp.zeros((pad,), jnp.int32)])
    dstp = jnp.concatenate([dst, jnp.full((pad,), N, jnp.int32)])
    idx2 = jnp.stack([srcp.reshape(nbt, K), dstp.reshape(nbt, K)], axis=1)
    idx2 = jnp.concatenate([idx2, jnp.zeros((1, 2, K), jnp.int32)], axis=0)
    ea128 = jnp.concatenate(
        [jnp.ones((E, 1), jnp.float32), edge_attr,
         jnp.zeros((E, D - 1 - ED), jnp.float32)], axis=1)
    ea128 = jnp.concatenate(
        [ea128, jnp.zeros((pad + K, D), jnp.float32)], axis=0)
    z128 = jnp.zeros((RPT, D), jnp.float32)

    Wa, ba, Ce = _tc_wprep(Wn, bn, We, be, Wm, bm)
    stats = _sc_segsum_rows(ea128, idx2, nb, z128)
    h, A = _tc_init(x, Wp, bp.reshape(1, D), Wa[0], ba[0])
    st0 = stats[0, :N, :16]
    st1 = stats[1, :N, :16]
    for l in range(L):
        acc = _sc_segsum_gather(A, idx2, z128)
        agg = jnp.concatenate([acc[0, :H], acc[1, :N - H]], axis=0)
        last = l == L - 1
        h, A = _tc_layer(
            h, agg, st0, st1, Ce[l],
            lng[l].reshape(1, D), lnb[l].reshape(1, D),
            None if last else Wa[l + 1], None if last else ba[l + 1], last)
    return h
